# baseline (speedup 1.0000x reference)
import functools

import jax
import jax.numpy as jnp
import numpy as np
from jax.experimental import pallas as pl
from jax.experimental.pallas import tpu as pltpu

_B = 128
_V = 100000
_BR = 8
_BV = 4096
_NV = (_V + _BV - 1) // _BV

_U32 = np.uint32
_TINY = np.float32(np.finfo(np.float32).tiny)


def _rotl(x, d):
    return jax.lax.shift_left(x, _U32(d)) | jax.lax.shift_right_logical(
        x, _U32(32 - d))


def _threefry_bits(flat_u32):
    ks0 = _U32(0)
    ks1 = _U32(42)
    ks2 = _U32(0x1BD11BDA ^ 42)
    ks = (ks0, ks1, ks2)
    rots = ((13, 15, 26, 6), (17, 29, 16, 24))
    x0 = jnp.full_like(flat_u32, ks0)
    x1 = flat_u32 + ks1
    for g in range(5):
        for r in rots[g % 2]:
            x0 = x0 + x1
            x1 = _rotl(x1, r)
            x1 = x0 ^ x1
        x0 = x0 + ks[(g + 1) % 3]
        x1 = x1 + ks[(g + 2) % 3] + _U32(g + 1)
    return x0 ^ x1


def _sample_kernel(logits_ref, t_ref, out_ref, val_ref, idx_ref):
    i = pl.program_id(0)
    j = pl.program_id(1)

    cols = jax.lax.broadcasted_iota(jnp.int32, (_BR, _BV), 1) + j * _BV
    rows = jax.lax.broadcasted_iota(jnp.int32, (_BR, _BV), 0) + i * _BR
    flat = (rows * _V + cols).astype(_U32)

    bits = _threefry_bits(flat)

    fb = jax.lax.shift_right_logical(bits, _U32(9)) | _U32(0x3F800000)
    floats = jax.lax.bitcast_convert_type(fb, jnp.float32) - np.float32(1.0)
    u = jnp.maximum(_TINY, floats * (np.float32(1.0) - _TINY) + _TINY)
    g = -jnp.log(-jnp.log(u))

    score = logits_ref[...] / t_ref[...] + g
    score = jnp.where(cols < _V, score, -jnp.inf)

    bm = jnp.max(score, axis=1, keepdims=True)
    bidx = jnp.min(jnp.where(score == bm, cols, np.int32(2**30)),
                   axis=1, keepdims=True)

    @pl.when(j == 0)
    def _init():
        val_ref[...] = jnp.full((_BR, 1), -jnp.inf, jnp.float32)
        idx_ref[...] = jnp.zeros((_BR, 1), jnp.int32)

    upd = bm > val_ref[...]
    val_ref[...] = jnp.where(upd, bm, val_ref[...])
    idx_ref[...] = jnp.where(upd, bidx, idx_ref[...])

    @pl.when(j == _NV - 1)
    def _emit():
        out_ref[...] = idx_ref[...]


@functools.partial(jax.jit, static_argnames=())
def kernel(logits, temperatures):
    logits = logits.astype(jnp.float32)
    t2 = temperatures.astype(jnp.float32).reshape(_B, 1)
    out = pl.pallas_call(
        _sample_kernel,
        grid=(_B // _BR, _NV),
        in_specs=[
            pl.BlockSpec((_BR, _BV), lambda i, j: (i, j)),
            pl.BlockSpec((_BR, 1), lambda i, j: (i, 0)),
        ],
        out_specs=pl.BlockSpec((_BR, 1), lambda i, j: (i, 0)),
        out_shape=jax.ShapeDtypeStruct((_B, 1), jnp.int32),
        scratch_shapes=[
            pltpu.VMEM((_BR, 1), jnp.float32),
            pltpu.VMEM((_BR, 1), jnp.int32),
        ],
    )(logits, t2)
    return out.reshape(_B)
